# baseline (speedup 1.0000x reference)
import numpy as np
import jax
import jax.numpy as jnp
from jax import lax
from jax.experimental import pallas as pl
from jax.experimental.pallas import tpu as pltpu
from jax.experimental.pallas import tpu_sc as plsc

_MAX_LEN = 200
_D = 64
_B = 1024
_NW = 32
_SEQ_PER_W = _B // _NW
_CH0 = 104
_CH1 = _MAX_LEN - _CH0


def _pe_table():
    position = np.expand_dims(np.arange(0, _MAX_LEN), axis=1).astype(np.float64)
    index = np.expand_dims(np.arange(0, _D, 2), axis=0).astype(np.float64)
    angle = position / np.power(1000.0, (index - index % 2) / float(_D))
    pe = np.zeros((_MAX_LEN, _D))
    pe[:, 0::2] = np.sin(angle)
    pe[:, 1::2] = np.cos(angle)
    return pe.astype(np.float32)


_PE = _pe_table()


def _body(idx_hbm, table_hbm, pe_hbm, out_hbm, pe_v, idx_v, rows_v, sem):
    wid = lax.axis_index("s") * 2 + lax.axis_index("c")
    pltpu.sync_copy(pe_hbm, pe_v)
    pltpu.sync_copy(
        idx_hbm.at[pl.ds(pl.multiple_of(wid * _SEQ_PER_W * _MAX_LEN, 8),
                         _SEQ_PER_W * _MAX_LEN)], idx_v)

    def one_seq(g, carry):
        b = wid * _SEQ_PER_W + g
        base = pl.multiple_of(g * _MAX_LEN, 8)
        base1 = pl.multiple_of(g * _MAX_LEN + _CH0, 8)
        c0 = pltpu.async_copy(table_hbm.at[idx_v.at[pl.ds(base, _CH0)]],
                              rows_v.at[pl.ds(0, _CH0)], sem)
        c1 = pltpu.async_copy(table_hbm.at[idx_v.at[pl.ds(base1, _CH1)]],
                              rows_v.at[pl.ds(_CH0, _CH1)], sem)
        c0.wait()
        c1.wait()

        def add_row(j, c):
            for k in range(_D // 16):
                s = pl.ds(k * 16, 16)
                rows_v[j, s] = rows_v[j, s] + pe_v[j, s]
            return c

        lax.fori_loop(0, _MAX_LEN, add_row, 0)
        pltpu.sync_copy(rows_v, out_hbm.at[b])
        return carry

    lax.fori_loop(0, _SEQ_PER_W, one_seq, 0)


def kernel(input, table):
    mesh = plsc.VectorSubcoreMesh(core_axis_name="c", subcore_axis_name="s")
    f = pl.kernel(
        _body,
        out_type=jax.ShapeDtypeStruct((_B, _MAX_LEN, _D), jnp.float32),
        mesh=mesh,
        compiler_params=pltpu.CompilerParams(use_tc_tiling_on_sc=False),
        scratch_types=[
            pltpu.VMEM((_MAX_LEN, _D), jnp.float32),
            pltpu.VMEM((_SEQ_PER_W * _MAX_LEN,), jnp.int32),
            pltpu.VMEM((_MAX_LEN, _D), jnp.float32),
            pltpu.SemaphoreType.DMA,
        ],
    )
    return f(input.reshape(-1), table, jnp.asarray(_PE))


# speedup vs baseline: 1.0019x; 1.0019x over previous
import numpy as np
import jax
import jax.numpy as jnp
from jax import lax
from jax.experimental import pallas as pl
from jax.experimental.pallas import tpu as pltpu
from jax.experimental.pallas import tpu_sc as plsc

_MAX_LEN = 200
_D = 64
_B = 1024
_NW = 32
_SEQ_PER_W = _B // _NW
_CH0 = 104
_CH1 = _MAX_LEN - _CH0


def _pe_table():
    position = np.expand_dims(np.arange(0, _MAX_LEN), axis=1).astype(np.float64)
    index = np.expand_dims(np.arange(0, _D, 2), axis=0).astype(np.float64)
    angle = position / np.power(1000.0, (index - index % 2) / float(_D))
    pe = np.zeros((_MAX_LEN, _D))
    pe[:, 0::2] = np.sin(angle)
    pe[:, 1::2] = np.cos(angle)
    return pe.astype(np.float32)


_PE = _pe_table()


def _body(idx_hbm, table_hbm, pe_hbm, out_hbm, pe_v, idx_v, rows_v, sem):
    wid = lax.axis_index("s") * 2 + lax.axis_index("c")
    pltpu.sync_copy(pe_hbm, pe_v)
    pltpu.sync_copy(
        idx_hbm.at[pl.ds(pl.multiple_of(wid * _SEQ_PER_W * _MAX_LEN, 8),
                         _SEQ_PER_W * _MAX_LEN)], idx_v)

    def one_seq(g, carry):
        b = wid * _SEQ_PER_W + g
        base = pl.multiple_of(g * _MAX_LEN, 8)
        base1 = pl.multiple_of(g * _MAX_LEN + _CH0, 8)
        c0 = pltpu.async_copy(table_hbm.at[idx_v.at[pl.ds(base, _CH0)]],
                              rows_v.at[pl.ds(0, _CH0)], sem)
        c1 = pltpu.async_copy(table_hbm.at[idx_v.at[pl.ds(base1, _CH1)]],
                              rows_v.at[pl.ds(_CH0, _CH1)], sem)
        c0.wait()
        c1.wait()

        @plsc.parallel_loop(0, _MAX_LEN, unroll=4)
        def add_row(j):
            for k in range(_D // 16):
                s = pl.ds(k * 16, 16)
                rows_v[j, s] = rows_v[j, s] + pe_v[j, s]
        pltpu.sync_copy(rows_v, out_hbm.at[b])
        return carry

    lax.fori_loop(0, _SEQ_PER_W, one_seq, 0)


def kernel(input, table):
    mesh = plsc.VectorSubcoreMesh(core_axis_name="c", subcore_axis_name="s")
    f = pl.kernel(
        _body,
        out_type=jax.ShapeDtypeStruct((_B, _MAX_LEN, _D), jnp.float32),
        mesh=mesh,
        compiler_params=pltpu.CompilerParams(use_tc_tiling_on_sc=False),
        scratch_types=[
            pltpu.VMEM((_MAX_LEN, _D), jnp.float32),
            pltpu.VMEM((_SEQ_PER_W * _MAX_LEN,), jnp.int32),
            pltpu.VMEM((_MAX_LEN, _D), jnp.float32),
            pltpu.SemaphoreType.DMA,
        ],
    )
    return f(input.reshape(-1), table, jnp.asarray(_PE))


# speedup vs baseline: 1.0660x; 1.0640x over previous
import numpy as np
import jax
import jax.numpy as jnp
from jax import lax
from jax.experimental import pallas as pl
from jax.experimental.pallas import tpu as pltpu
from jax.experimental.pallas import tpu_sc as plsc

_MAX_LEN = 200
_D = 64
_B = 1024
_NW = 32
_SEQ_PER_W = _B // _NW
_CH0 = 104
_CH1 = _MAX_LEN - _CH0
_NBUF = 4


def _pe_table():
    position = np.expand_dims(np.arange(0, _MAX_LEN), axis=1).astype(np.float64)
    index = np.expand_dims(np.arange(0, _D, 2), axis=0).astype(np.float64)
    angle = position / np.power(1000.0, (index - index % 2) / float(_D))
    pe = np.zeros((_MAX_LEN, _D))
    pe[:, 0::2] = np.sin(angle)
    pe[:, 1::2] = np.cos(angle)
    return pe.astype(np.float32)


_PE = _pe_table()


def _body(idx_hbm, table_hbm, pe_hbm, out_hbm, pe_v, idx_v, rows_v, *sems):
    g_sems = sems[:_NBUF]
    s_sems = sems[_NBUF:]
    wid = lax.axis_index("s") * 2 + lax.axis_index("c")
    seq0 = wid * _SEQ_PER_W
    pltpu.sync_copy(pe_hbm, pe_v)
    pltpu.sync_copy(
        idx_hbm.at[pl.ds(pl.multiple_of(seq0 * _MAX_LEN, 8),
                         _SEQ_PER_W * _MAX_LEN)], idx_v)

    def fire_gather(g, p):
        base = pl.multiple_of(g * _MAX_LEN, 8)
        base1 = pl.multiple_of(g * _MAX_LEN + _CH0, 8)
        pltpu.async_copy(table_hbm.at[idx_v.at[pl.ds(base, _CH0)]],
                         rows_v.at[p, pl.ds(0, _CH0)], g_sems[p])
        pltpu.async_copy(table_hbm.at[idx_v.at[pl.ds(base1, _CH1)]],
                         rows_v.at[p, pl.ds(_CH0, _CH1)], g_sems[p])

    def drain_gather(p):
        pltpu.make_async_copy(table_hbm.at[pl.ds(0, _MAX_LEN)],
                              rows_v.at[p], g_sems[p]).wait()

    def drain_store(p):
        pltpu.make_async_copy(rows_v.at[p], out_hbm.at[0], s_sems[p]).wait()

    for p in range(_NBUF - 1):
        fire_gather(p, p)

    def step(h, carry):
        for p in range(_NBUF):
            g = h * _NBUF + p
            drain_gather(p)

            @plsc.parallel_loop(0, _MAX_LEN, unroll=4)
            def add_row(j):
                for k in range(_D // 16):
                    s = pl.ds(k * 16, 16)
                    rows_v[p, j, s] = rows_v[p, j, s] + pe_v[j, s]

            pltpu.async_copy(rows_v.at[p], out_hbm.at[seq0 + g], s_sems[p])

            q = (p + _NBUF - 1) % _NBUF

            @pl.when(g >= 1)
            def _():
                drain_store(q)

            @pl.when(g + _NBUF - 1 < _SEQ_PER_W)
            def _():
                fire_gather(g + _NBUF - 1, q)
        return carry

    lax.fori_loop(0, _SEQ_PER_W // _NBUF, step, 0)
    drain_store(_NBUF - 1)


def kernel(input, table):
    mesh = plsc.VectorSubcoreMesh(core_axis_name="c", subcore_axis_name="s")
    f = pl.kernel(
        _body,
        out_type=jax.ShapeDtypeStruct((_B, _MAX_LEN, _D), jnp.float32),
        mesh=mesh,
        compiler_params=pltpu.CompilerParams(use_tc_tiling_on_sc=False),
        scratch_types=[
            pltpu.VMEM((_MAX_LEN, _D), jnp.float32),
            pltpu.VMEM((_SEQ_PER_W * _MAX_LEN,), jnp.int32),
            pltpu.VMEM((_NBUF, _MAX_LEN, _D), jnp.float32),
        ] + [pltpu.SemaphoreType.DMA] * (2 * _NBUF),
    )
    return f(input.reshape(-1), table, jnp.asarray(_PE))


# speedup vs baseline: 1.1126x; 1.0437x over previous
import numpy as np
import jax
import jax.numpy as jnp
from jax import lax
from jax.experimental import pallas as pl
from jax.experimental.pallas import tpu as pltpu
from jax.experimental.pallas import tpu_sc as plsc

_MAX_LEN = 200
_D = 64
_B = 1024
_NW = 32
_SEQ_PER_W = _B // _NW
_CH0 = 104
_CH1 = _MAX_LEN - _CH0
_NBUF = 4


def _pe_table():
    position = np.expand_dims(np.arange(0, _MAX_LEN), axis=1).astype(np.float64)
    index = np.expand_dims(np.arange(0, _D, 2), axis=0).astype(np.float64)
    angle = position / np.power(1000.0, (index - index % 2) / float(_D))
    pe = np.zeros((_MAX_LEN, _D))
    pe[:, 0::2] = np.sin(angle)
    pe[:, 1::2] = np.cos(angle)
    return pe.astype(np.float32)


_PE = _pe_table()


def _body(idx_hbm, table_hbm, pe_hbm, out_hbm, pe_v, idx_v, rows_v, *sems):
    g_sems = sems[:_NBUF]
    s_sems = sems[_NBUF:]
    wid = lax.axis_index("s") * 2 + lax.axis_index("c")
    seq0 = wid * _SEQ_PER_W
    pltpu.sync_copy(pe_hbm, pe_v)
    pltpu.sync_copy(
        idx_hbm.at[pl.ds(pl.multiple_of(seq0 * _MAX_LEN, 8),
                         _SEQ_PER_W * _MAX_LEN)], idx_v)

    def fire_gather(g, p):
        base = pl.multiple_of(g * _MAX_LEN, 8)
        base1 = pl.multiple_of(g * _MAX_LEN + _CH0, 8)
        pltpu.async_copy(table_hbm.at[idx_v.at[pl.ds(base, _CH0)]],
                         rows_v.at[p, pl.ds(0, _CH0)], g_sems[p])
        pltpu.async_copy(table_hbm.at[idx_v.at[pl.ds(base1, _CH1)]],
                         rows_v.at[p, pl.ds(_CH0, _CH1)], g_sems[p])

    def drain_gather(p):
        pltpu.make_async_copy(table_hbm.at[pl.ds(0, _MAX_LEN)],
                              rows_v.at[p], g_sems[p]).wait()

    def drain_store(p):
        pltpu.make_async_copy(rows_v.at[p, :, pl.ds(0, _D)],
                              out_hbm.at[pl.ds(0, _MAX_LEN)],
                              s_sems[p]).wait()

    for p in range(_NBUF - 1):
        fire_gather(p, p)

    def step(h, carry):
        for p in range(_NBUF):
            g = h * _NBUF + p
            drain_gather(p)

            @plsc.parallel_loop(0, _MAX_LEN, unroll=4)
            def add_row(j):
                for k in range(_D // 16):
                    s = pl.ds(k * 16, 16)
                    rows_v[p, j, s] = rows_v[p, j, s] + pe_v[j, s]

            ob = pl.multiple_of((seq0 + g) * _MAX_LEN, 8)
            pltpu.async_copy(rows_v.at[p, :, pl.ds(0, _D)],
                             out_hbm.at[pl.ds(ob, _MAX_LEN)], s_sems[p])

            q = (p + _NBUF - 1) % _NBUF

            @pl.when(g >= 1)
            def _():
                drain_store(q)

            @pl.when(g + _NBUF - 1 < _SEQ_PER_W)
            def _():
                fire_gather(g + _NBUF - 1, q)
        return carry

    lax.fori_loop(0, _SEQ_PER_W // _NBUF, step, 0)
    drain_store(_NBUF - 1)


def kernel(input, table):
    mesh = plsc.VectorSubcoreMesh(core_axis_name="c", subcore_axis_name="s")
    f = pl.kernel(
        _body,
        out_type=jax.ShapeDtypeStruct((_B * _MAX_LEN, _D), jnp.float32),
        mesh=mesh,
        compiler_params=pltpu.CompilerParams(use_tc_tiling_on_sc=False),
        scratch_types=[
            pltpu.VMEM((_MAX_LEN, _D), jnp.float32),
            pltpu.VMEM((_SEQ_PER_W * _MAX_LEN,), jnp.int32),
            pltpu.VMEM((_NBUF, _MAX_LEN, 2 * _D), jnp.float32),
        ] + [pltpu.SemaphoreType.DMA] * (2 * _NBUF),
    )
    table128 = jnp.pad(table, ((0, 0), (0, _D)))
    out = f(input.reshape(-1), table128, jnp.asarray(_PE))
    return out.reshape(_B, _MAX_LEN, _D)
